# Initial kernel scaffold; baseline (speedup 1.0000x reference)
import jax
import jax.numpy as jnp
from jax import lax
from jax.experimental import pallas as pl
from jax.experimental.pallas import tpu as pltpu
from jax.experimental.pallas import tpu_sc as plsc

N = 16384
DF = 64
DO = 128
K = 32
R2 = 0.2 * 0.2
NC = 2
NS = 16
NW = NC * NS
QPW = N // NW
HCAP = 320


def _ab_body(xp_ref, pp_ref, w_ref, b_ref, a_ref, bout_ref):
    w = w_ref[...]
    a_ref[...] = jnp.dot(xp_ref[...], w, preferred_element_type=jnp.float32) + b_ref[...]
    bout_ref[...] = jnp.dot(pp_ref[...], w, preferred_element_type=jnp.float32)


def _ab(xp, pp, w1p, b1):
    t = 512
    return pl.pallas_call(
        _ab_body,
        grid=(N // t,),
        in_specs=[
            pl.BlockSpec((t, 128), lambda i: (i, 0)),
            pl.BlockSpec((t, 128), lambda i: (i, 0)),
            pl.BlockSpec((128, DF), lambda i: (0, 0)),
            pl.BlockSpec((1, DF), lambda i: (0, 0)),
        ],
        out_specs=[
            pl.BlockSpec((t, DF), lambda i: (i, 0)),
            pl.BlockSpec((t, DF), lambda i: (i, 0)),
        ],
        out_shape=[
            jax.ShapeDtypeStruct((N, DF), jnp.float32),
            jax.ShapeDtypeStruct((N, DF), jnp.float32),
        ],
    )(xp, pp, w1p, b1)


def _sc_body(px_hbm, py_hbm, pz_hbm, bat_hbm, a_hbm, g_hbm,
             px_v, py_v, pz_v, p2_v, bat_v, seg_v, hd_v, hj_v,
             selb_v, sel_v, rows_v, sem):
    wid = lax.axis_index("s") * NC + lax.axis_index("c")
    pltpu.sync_copy(px_hbm, px_v)
    pltpu.sync_copy(py_hbm, py_v)
    pltpu.sync_copy(pz_hbm, pz_v)
    pltpu.sync_copy(bat_hbm, bat_v)
    iot = lax.iota(jnp.int32, 16)

    def _pre(c, _):
        s = c * 16
        vx = px_v[pl.ds(s, 16)]
        vy = py_v[pl.ds(s, 16)]
        vz = pz_v[pl.ds(s, 16)]
        p2_v[pl.ds(s, 16)] = (vx * vx + vy * vy) + vz * vz
        jg = iot + s
        bv = bat_v[pl.ds(s, 16)]
        prevv = plsc.load_gather(bat_v, [jnp.maximum(jg - 1, 0)])
        nxtv = plsc.load_gather(bat_v, [jnp.minimum(jg + 1, N - 1)])
        isf = (bv != prevv) | (jg == 0)
        isl = (bv != nxtv) | (jg == N - 1)
        plsc.store_scatter(seg_v, [bv], jg, mask=isf)
        plsc.store_scatter(seg_v, [bv + 17], jg + 1, mask=isl)
        return 0

    lax.fori_loop(0, N // 16, _pre, 0)

    q0 = wid * QPW

    def _q(qi, _):
        i = q0 + qi
        vb = bat_v[i]
        lo = seg_v[vb]
        hi = seg_v[vb + 17]
        qx = px_v[i]
        qy = py_v[i]
        qz = pz_v[i]
        q2 = p2_v[i]
        c0 = lax.div(lo, 16)
        c1 = lax.div(hi + 15, 16)

        def _scan(c, cur):
            s = c * 16
            jg = iot + s
            vx = px_v[pl.ds(s, 16)]
            vy = py_v[pl.ds(s, 16)]
            vz = pz_v[pl.ds(s, 16)]
            vp2 = p2_v[pl.ds(s, 16)]
            d2 = jnp.maximum((q2 + vp2) - 2.0 * ((qx * vx + qy * vy) + qz * vz), 0.0)
            m = (d2 <= R2) & (jg >= lo) & (jg < hi) & (cur < HCAP - 16)
            plsc.store_compressed(hd_v.at[pl.ds(cur, 16)], d2, mask=m)
            plsc.store_compressed(hj_v.at[pl.ds(cur, 16)], jg, mask=m)
            return cur + jnp.sum(m.astype(jnp.int32))

        nh = lax.fori_loop(c0, c1, _scan, 0)
        nch = lax.div(nh + 15, 16)

        def _cond(st):
            return st > K

        def _drop(st):
            def _mx(c, mx):
                dv = hd_v[pl.ds(c * 16, 16)]
                lm = (iot + c * 16) < nh
                return jnp.maximum(mx, jnp.max(jnp.where(lm, dv, -1.0)))

            mval = lax.fori_loop(0, nch, _mx, -1.0)

            def _pw(c, p):
                dv = hd_v[pl.ds(c * 16, 16)]
                jgl = iot + c * 16
                lm = (jgl < nh) & (dv == mval)
                return jnp.maximum(p, jnp.max(jnp.where(lm, jgl, -1)))

            ppos = lax.fori_loop(0, nch, _pw, -1)
            hd_v[ppos] = -1.0
            return st - 1

        lax.while_loop(_cond, _drop, nh)

        selb_v[pl.ds(0, 16)] = jnp.full((16,), i, jnp.int32)
        selb_v[pl.ds(16, 16)] = jnp.full((16,), i, jnp.int32)

        def _cp(c, cur):
            dv = hd_v[pl.ds(c * 16, 16)]
            jv = hj_v[pl.ds(c * 16, 16)]
            lm = ((iot + c * 16) < nh) & (dv >= 0.0)
            plsc.store_compressed(selb_v.at[pl.ds(cur, 16)], jv, mask=lm)
            return cur + jnp.sum(lm.astype(jnp.int32))

        lax.fori_loop(0, nch, _cp, 0)
        sel_v[pl.ds(0, 16)] = selb_v[pl.ds(0, 16)]
        sel_v[pl.ds(16, 16)] = selb_v[pl.ds(16, 16)]
        pltpu.async_copy(a_hbm.at[sel_v], rows_v, sem).wait()
        pltpu.sync_copy(rows_v, g_hbm.at[pl.ds(i * K, K)])
        return 0

    lax.fori_loop(0, QPW, _q, 0)


def _sc(px, py, pz, bat, a):
    mesh = plsc.VectorSubcoreMesh(core_axis_name="c", subcore_axis_name="s")
    f = pl.kernel(
        _sc_body,
        out_type=jax.ShapeDtypeStruct((N * K, DF), jnp.float32),
        mesh=mesh,
        scratch_types=[
            pltpu.VMEM((N,), jnp.float32),
            pltpu.VMEM((N,), jnp.float32),
            pltpu.VMEM((N,), jnp.float32),
            pltpu.VMEM((N,), jnp.float32),
            pltpu.VMEM((N,), jnp.int32),
            pltpu.VMEM((34,), jnp.int32),
            pltpu.VMEM((HCAP,), jnp.float32),
            pltpu.VMEM((HCAP,), jnp.int32),
            pltpu.VMEM((48,), jnp.int32),
            pltpu.VMEM((K,), jnp.int32),
            pltpu.VMEM((K, DF), jnp.float32),
            pltpu.SemaphoreType.DMA,
        ],
    )
    return f(px, py, pz, bat, a)


QT = 256


def _mlp_body(g_ref, b_ref, w2_ref, b2_ref, w3_ref, b3_ref, o_ref):
    g = g_ref[...].reshape(QT, K, DF)
    h1 = jnp.maximum(g - b_ref[...][:, None, :], 0.0).reshape(QT * K, DF)
    h2 = jnp.maximum(
        jnp.dot(h1, w2_ref[...], preferred_element_type=jnp.float32) + b2_ref[...], 0.0)
    h3 = jnp.maximum(
        jnp.dot(h2, w3_ref[...], preferred_element_type=jnp.float32) + b3_ref[...], 0.0)
    o_ref[...] = jnp.max(h3.reshape(QT, K, DO), axis=1)


def _mlp(g, b, w2, b2, w3, b3):
    return pl.pallas_call(
        _mlp_body,
        grid=(N // QT,),
        in_specs=[
            pl.BlockSpec((QT * K, DF), lambda i: (i, 0)),
            pl.BlockSpec((QT, DF), lambda i: (i, 0)),
            pl.BlockSpec((DF, DF), lambda i: (0, 0)),
            pl.BlockSpec((1, DF), lambda i: (0, 0)),
            pl.BlockSpec((DF, DO), lambda i: (0, 0)),
            pl.BlockSpec((1, DO), lambda i: (0, 0)),
        ],
        out_specs=pl.BlockSpec((QT, DO), lambda i: (i, 0)),
        out_shape=jax.ShapeDtypeStruct((N, DO), jnp.float32),
    )(g, b, w2, b2, w3, b3)


@jax.jit
def kernel(x, pos, batch, W1, b1, W2, b2, W3, b3):
    zpad = jnp.zeros((N, 61), jnp.float32)
    xp = jnp.concatenate([x, pos, zpad], axis=1)
    pp = jnp.concatenate([jnp.zeros((N, DF), jnp.float32), pos, zpad], axis=1)
    w1p = jnp.concatenate([W1, jnp.zeros((61, DF), jnp.float32)], axis=0)
    a, b = _ab(xp, pp, w1p, b1.reshape(1, DF))
    g = _sc(pos[:, 0], pos[:, 1], pos[:, 2], batch, a)
    out = _mlp(g, b, W2, b2.reshape(1, DF), W3, b3.reshape(1, DO))
    return (out, pos, batch)


# speedup vs baseline: 20.0441x; 20.0441x over previous
import jax
import jax.numpy as jnp
from jax import lax
from jax.experimental import pallas as pl
from jax.experimental.pallas import tpu as pltpu
from jax.experimental.pallas import tpu_sc as plsc

N = 16384
DF = 64
DO = 128
K = 32
R2 = 0.2 * 0.2
NC = 2
NS = 16
NW = NC * NS
QPW = N // NW
HCAP = 512


def _xw_body(x_ref, w_ref, b_ref, o_ref):
    o_ref[...] = jnp.dot(x_ref[...], w_ref[...],
                         preferred_element_type=jnp.float32) + b_ref[...]


def _xw(x, w, b):
    t = 512
    return pl.pallas_call(
        _xw_body,
        grid=(N // t,),
        in_specs=[
            pl.BlockSpec((t, DF), lambda i: (i, 0)),
            pl.BlockSpec((DF, DF), lambda i: (0, 0)),
            pl.BlockSpec((1, DF), lambda i: (0, 0)),
        ],
        out_specs=pl.BlockSpec((t, DF), lambda i: (i, 0)),
        out_shape=jax.ShapeDtypeStruct((N, DF), jnp.float32),
    )(x, w, b)


def _sload(ref, idx):
    return plsc.load_gather(ref, [jnp.full((16,), idx, jnp.int32)])[0]


def _rbf16(v):
    u = plsc.bitcast(v, jnp.int32)
    r = (u + 0x7FFF + ((u >> 16) & 1)) & jnp.int32(-65536)
    return plsc.bitcast(r, jnp.float32)


def _sc_body(px_hbm, py_hbm, pz_hbm, bat_hbm, xw_hbm, gx_hbm, rel_hbm,
             px_v, py_v, pz_v, p2_v, bat_v, seg_v, hd_v, hj_v,
             selb_v, sel_v, relf_v, rows_v, sem):
    wid = lax.axis_index("s") * NC + lax.axis_index("c")
    pltpu.sync_copy(px_hbm, px_v)
    pltpu.sync_copy(py_hbm, py_v)
    pltpu.sync_copy(pz_hbm, pz_v)
    pltpu.sync_copy(bat_hbm, bat_v)
    iot = lax.iota(jnp.int32, 16)

    zero16 = jnp.zeros((16,), jnp.float32)
    for t in range(8):
        relf_v[pl.ds(t * 16, 16)] = zero16

    def _pre(c, _):
        s = c * 16
        vx = px_v[pl.ds(s, 16)]
        vy = py_v[pl.ds(s, 16)]
        vz = pz_v[pl.ds(s, 16)]
        p2_v[pl.ds(s, 16)] = (vx * vx + vy * vy) + vz * vz
        jg = iot + s
        bv = bat_v[pl.ds(s, 16)]
        prevv = plsc.load_gather(bat_v, [jnp.maximum(jg - 1, 0)])
        nxtv = plsc.load_gather(bat_v, [jnp.minimum(jg + 1, N - 1)])
        isf = (bv != prevv) | (jg == 0)
        isl = (bv != nxtv) | (jg == N - 1)
        plsc.store_scatter(seg_v, [bv], jg, mask=isf)
        plsc.store_scatter(seg_v, [bv + 16], jg + 1, mask=isl)
        return 0

    lax.fori_loop(0, N // 16, _pre, 0)

    q0 = wid * QPW

    def _q(qi, _):
        i = q0 + qi
        vb = _sload(bat_v, i)
        lo = _sload(seg_v, vb)
        hi = _sload(seg_v, vb + 16)
        ii = jnp.full((16,), i, jnp.int32)
        qxv = plsc.load_gather(px_v, [ii])
        qyv = plsc.load_gather(py_v, [ii])
        qzv = plsc.load_gather(pz_v, [ii])
        qxe = qxv[0]
        qye = qyv[0]
        qze = qzv[0]
        q2 = _sload(p2_v, i)
        qx = _rbf16(qxv)[0]
        qy = _rbf16(qyv)[0]
        qz = _rbf16(qzv)[0]
        c0 = lax.div(lo, 16)
        c1 = lax.div(hi + 15, 16)

        def _scan(c, cur):
            s = c * 16
            jg = iot + s
            vx = _rbf16(px_v[pl.ds(s, 16)])
            vy = _rbf16(py_v[pl.ds(s, 16)])
            vz = _rbf16(pz_v[pl.ds(s, 16)])
            vp2 = p2_v[pl.ds(s, 16)]
            d2 = jnp.maximum((q2 + vp2) - 2.0 * ((qx * vx + qy * vy) + qz * vz), 0.0)
            m = (d2 <= R2) & (jg >= lo) & (jg < hi) & (cur < HCAP - 16)
            plsc.store_compressed(hd_v.at[pl.ds(cur, 16)], d2, mask=m)
            plsc.store_compressed(hj_v.at[pl.ds(cur, 16)], jg, mask=m)
            return cur + jnp.sum(m.astype(jnp.int32))

        nh = lax.fori_loop(c0, c1, _scan, 0)
        nch = lax.div(nh + 15, 16)

        def _cond(st):
            return st > K

        def _drop(st):
            def _mx(c, mx):
                dv = hd_v[pl.ds(c * 16, 16)]
                lm = (iot + c * 16) < nh
                return jnp.maximum(mx, jnp.max(jnp.where(lm, dv, -1.0)))

            mval = lax.fori_loop(0, nch, _mx, -1.0)

            def _pw(c, p):
                dv = hd_v[pl.ds(c * 16, 16)]
                jgl = iot + c * 16
                lm = (jgl < nh) & (dv == mval)
                return jnp.maximum(p, jnp.max(jnp.where(lm, jgl, -1)))

            ppos = lax.fori_loop(0, nch, _pw, -1)
            plsc.store_scatter(hd_v, [jnp.full((16,), ppos, jnp.int32)],
                               jnp.full((16,), -1.0, jnp.float32), mask=iot == 0)
            return st - 1

        lax.while_loop(_cond, _drop, nh)

        selb_v[pl.ds(0, 16)] = jnp.full((16,), i, jnp.int32)
        selb_v[pl.ds(16, 16)] = jnp.full((16,), i, jnp.int32)

        def _cp(c, cur):
            dv = hd_v[pl.ds(c * 16, 16)]
            jv = hj_v[pl.ds(c * 16, 16)]
            lm = ((iot + c * 16) < nh) & (dv >= 0.0)
            plsc.store_compressed(selb_v.at[pl.ds(cur, 16)], jv, mask=lm)
            return cur + jnp.sum(lm.astype(jnp.int32))

        lax.fori_loop(0, nch, _cp, 0)
        sel_v[pl.ds(0, 16)] = selb_v[pl.ds(0, 16)]
        sel_v[pl.ds(16, 16)] = selb_v[pl.ds(16, 16)]

        for h in range(2):
            selc = sel_v[pl.ds(h * 16, 16)]
            base = (iot + h * 16) * 4
            rx = plsc.load_gather(px_v, [selc]) - qxe
            ry = plsc.load_gather(py_v, [selc]) - qye
            rz = plsc.load_gather(pz_v, [selc]) - qze
            plsc.store_scatter(relf_v, [base], rx)
            plsc.store_scatter(relf_v, [base + 1], ry)
            plsc.store_scatter(relf_v, [base + 2], rz)

        pltpu.async_copy(xw_hbm.at[sel_v], rows_v, sem).wait()
        pltpu.sync_copy(rows_v, gx_hbm.at[pl.ds(i * K, K)])
        pltpu.sync_copy(relf_v, rel_hbm.at[pl.ds(i * K * 4, K * 4)])
        return 0

    lax.fori_loop(0, QPW, _q, 0)


def _sc(px, py, pz, bat, xw):
    mesh = plsc.VectorSubcoreMesh(core_axis_name="c", subcore_axis_name="s")
    f = pl.kernel(
        _sc_body,
        out_type=[
            jax.ShapeDtypeStruct((N * K, DF), jnp.float32),
            jax.ShapeDtypeStruct((N * K * 4,), jnp.float32),
        ],
        mesh=mesh,
        compiler_params=pltpu.CompilerParams(
            needs_layout_passes=False, use_tc_tiling_on_sc=False),
        scratch_types=[
            pltpu.VMEM((N,), jnp.float32),
            pltpu.VMEM((N,), jnp.float32),
            pltpu.VMEM((N,), jnp.float32),
            pltpu.VMEM((N,), jnp.float32),
            pltpu.VMEM((N,), jnp.int32),
            pltpu.VMEM((34,), jnp.int32),
            pltpu.VMEM((HCAP,), jnp.float32),
            pltpu.VMEM((HCAP,), jnp.int32),
            pltpu.VMEM((48,), jnp.int32),
            pltpu.VMEM((K,), jnp.int32),
            pltpu.VMEM((K * 4,), jnp.float32),
            pltpu.VMEM((K, DF), jnp.float32),
            pltpu.SemaphoreType.DMA,
        ],
    )
    return f(px, py, pz, bat, xw)


QT = 256


def _mlp_body(gx_ref, rel_ref, w1r_ref, w2_ref, b2_ref, w3_ref, b3_ref, o_ref):
    h0 = gx_ref[...] + jnp.dot(rel_ref[...], w1r_ref[...],
                               preferred_element_type=jnp.float32)
    h1 = jnp.maximum(h0, 0.0)
    h2 = jnp.maximum(
        jnp.dot(h1, w2_ref[...], preferred_element_type=jnp.float32) + b2_ref[...], 0.0)
    h3 = jnp.maximum(
        jnp.dot(h2, w3_ref[...], preferred_element_type=jnp.float32) + b3_ref[...], 0.0)
    o_ref[...] = jnp.max(h3.reshape(QT, K, DO), axis=1)


def _mlp(gx, rel, w1r, w2, b2, w3, b3):
    return pl.pallas_call(
        _mlp_body,
        grid=(N // QT,),
        in_specs=[
            pl.BlockSpec((QT * K, DF), lambda i: (i, 0)),
            pl.BlockSpec((QT * K, 4), lambda i: (i, 0)),
            pl.BlockSpec((4, DF), lambda i: (0, 0)),
            pl.BlockSpec((DF, DF), lambda i: (0, 0)),
            pl.BlockSpec((1, DF), lambda i: (0, 0)),
            pl.BlockSpec((DF, DO), lambda i: (0, 0)),
            pl.BlockSpec((1, DO), lambda i: (0, 0)),
        ],
        out_specs=pl.BlockSpec((QT, DO), lambda i: (i, 0)),
        out_shape=jax.ShapeDtypeStruct((N, DO), jnp.float32),
    )(gx, rel, w1r, w2, b2, w3, b3)


@jax.jit
def kernel(x, pos, batch, W1, b1, W2, b2, W3, b3):
    xw = _xw(x, W1[:DF], b1.reshape(1, DF))
    gx, relf = _sc(pos[:, 0], pos[:, 1], pos[:, 2], batch, xw)
    w1r = jnp.concatenate([W1[DF:], jnp.zeros((1, DF), jnp.float32)], axis=0)
    out = _mlp(gx, relf.reshape(N * K, 4), w1r, W2,
               b2.reshape(1, DF), W3, b3.reshape(1, DO))
    return (out, pos, batch)


# speedup vs baseline: 20.0796x; 1.0018x over previous
import jax
import jax.numpy as jnp
from jax import lax
from jax.experimental import pallas as pl
from jax.experimental.pallas import tpu as pltpu
from jax.experimental.pallas import tpu_sc as plsc

N = 16384
DF = 64
DO = 128
K = 32
R2 = 0.2 * 0.2
NC = 2
NS = 16
NW = NC * NS
QPW = N // NW
HCAP = 512


def _xw_body(x_ref, w_ref, b_ref, o_ref):
    o_ref[...] = jnp.dot(x_ref[...], w_ref[...],
                         preferred_element_type=jnp.float32) + b_ref[...]


def _xw(x, w, b):
    t = 512
    return pl.pallas_call(
        _xw_body,
        grid=(N // t,),
        in_specs=[
            pl.BlockSpec((t, DF), lambda i: (i, 0)),
            pl.BlockSpec((DF, DF), lambda i: (0, 0)),
            pl.BlockSpec((1, DF), lambda i: (0, 0)),
        ],
        out_specs=pl.BlockSpec((t, DF), lambda i: (i, 0)),
        out_shape=jax.ShapeDtypeStruct((N, DF), jnp.float32),
    )(x, w, b)


def _sload(ref, idx):
    return plsc.load_gather(ref, [jnp.full((16,), idx, jnp.int32)])[0]


def _rbf16_bits(v):
    u = plsc.bitcast(v, jnp.int32)
    return (u + 0x7FFF + ((u >> 16) & 1)) & jnp.int32(-65536)


def _rbf16(v):
    return plsc.bitcast(_rbf16_bits(v), jnp.float32)


def _sc_body(px_hbm, py_hbm, pz_hbm, bat_hbm, xw_hbm, gx_hbm, rel_hbm,
             px_v, py_v, pz_v, p2_v, bat_v, pxy_v, pzp_v,
             seg_v, hd_v, hj_v, selb_v, sel_v, relf_v, rows_v, sem):
    wid = lax.axis_index("s") * NC + lax.axis_index("c")
    pltpu.sync_copy(px_hbm, px_v)
    pltpu.sync_copy(py_hbm, py_v)
    pltpu.sync_copy(pz_hbm, pz_v)
    pltpu.sync_copy(bat_hbm, bat_v)
    iot = lax.iota(jnp.int32, 16)

    zero16 = jnp.zeros((16,), jnp.float32)
    for t in range(8):
        relf_v[pl.ds(t * 16, 16)] = zero16

    def _pre(c, _):
        s = c * 16
        vx = px_v[pl.ds(s, 16)]
        vy = py_v[pl.ds(s, 16)]
        vz = pz_v[pl.ds(s, 16)]
        p2_v[pl.ds(s, 16)] = (vx * vx + vy * vy) + vz * vz
        pxy_v[pl.ds(s, 16)] = _rbf16_bits(vx) | lax.shift_right_logical(
            _rbf16_bits(vy), 16)
        pzp_v[pl.ds(s, 16)] = _rbf16(vz)
        jg = iot + s
        bv = bat_v[pl.ds(s, 16)]
        prevv = plsc.load_gather(bat_v, [jnp.maximum(jg - 1, 0)])
        nxtv = plsc.load_gather(bat_v, [jnp.minimum(jg + 1, N - 1)])
        isf = (bv != prevv) | (jg == 0)
        isl = (bv != nxtv) | (jg == N - 1)
        plsc.store_scatter(seg_v, [bv], jg, mask=isf)
        plsc.store_scatter(seg_v, [bv + 16], jg + 1, mask=isl)
        return 0

    lax.fori_loop(0, N // 16, _pre, 0)

    q0 = wid * QPW

    def _q(qi, _):
        i = q0 + qi
        vb = _sload(bat_v, i)
        lo = _sload(seg_v, vb)
        hi = _sload(seg_v, vb + 16)
        ii = jnp.full((16,), i, jnp.int32)
        qxv = plsc.load_gather(px_v, [ii])
        qyv = plsc.load_gather(py_v, [ii])
        qzv = plsc.load_gather(pz_v, [ii])
        qxe = qxv[0]
        qye = qyv[0]
        qze = qzv[0]
        q2 = _sload(p2_v, i)
        qw = plsc.load_gather(pxy_v, [ii])
        qx = plsc.bitcast(qw & jnp.int32(-65536), jnp.float32)[0]
        qy = plsc.bitcast(qw << 16, jnp.float32)[0]
        qz = plsc.load_gather(pzp_v, [ii])[0]
        c0 = lax.div(lo, 16)
        c1 = lax.div(hi + 15, 16)

        def _scan(c, cur):
            s = c * 16
            jg = iot + s
            w = pxy_v[pl.ds(s, 16)]
            vx = plsc.bitcast(w & jnp.int32(-65536), jnp.float32)
            vy = plsc.bitcast(w << 16, jnp.float32)
            vz = pzp_v[pl.ds(s, 16)]
            vp2 = p2_v[pl.ds(s, 16)]
            d2 = jnp.maximum((q2 + vp2) - 2.0 * ((qx * vx + qy * vy) + qz * vz), 0.0)
            m = (d2 <= R2) & (jg >= lo) & (jg < hi) & (cur < HCAP - 16)
            plsc.store_compressed(hd_v.at[pl.ds(cur, 16)], d2, mask=m)
            plsc.store_compressed(hj_v.at[pl.ds(cur, 16)], jg, mask=m)
            return cur + jnp.sum(m.astype(jnp.int32))

        nh = lax.fori_loop(c0, c1, _scan, 0)
        nch = lax.div(nh + 15, 16)

        def _cond(st):
            return st > K

        def _drop(st):
            def _mx(c, mx):
                dv = hd_v[pl.ds(c * 16, 16)]
                lm = (iot + c * 16) < nh
                return jnp.maximum(mx, jnp.max(jnp.where(lm, dv, -1.0)))

            mval = lax.fori_loop(0, nch, _mx, -1.0)

            def _pw(c, p):
                dv = hd_v[pl.ds(c * 16, 16)]
                jgl = iot + c * 16
                lm = (jgl < nh) & (dv == mval)
                return jnp.maximum(p, jnp.max(jnp.where(lm, jgl, -1)))

            ppos = lax.fori_loop(0, nch, _pw, -1)
            plsc.store_scatter(hd_v, [jnp.full((16,), ppos, jnp.int32)],
                               jnp.full((16,), -1.0, jnp.float32), mask=iot == 0)
            return st - 1

        lax.while_loop(_cond, _drop, nh)

        selb_v[pl.ds(0, 16)] = jnp.full((16,), i, jnp.int32)
        selb_v[pl.ds(16, 16)] = jnp.full((16,), i, jnp.int32)

        def _cp(c, cur):
            dv = hd_v[pl.ds(c * 16, 16)]
            jv = hj_v[pl.ds(c * 16, 16)]
            lm = ((iot + c * 16) < nh) & (dv >= 0.0)
            plsc.store_compressed(selb_v.at[pl.ds(cur, 16)], jv, mask=lm)
            return cur + jnp.sum(lm.astype(jnp.int32))

        lax.fori_loop(0, nch, _cp, 0)
        sel_v[pl.ds(0, 16)] = selb_v[pl.ds(0, 16)]
        sel_v[pl.ds(16, 16)] = selb_v[pl.ds(16, 16)]

        for h in range(2):
            selc = sel_v[pl.ds(h * 16, 16)]
            base = (iot + h * 16) * 4
            rx = plsc.load_gather(px_v, [selc]) - qxe
            ry = plsc.load_gather(py_v, [selc]) - qye
            rz = plsc.load_gather(pz_v, [selc]) - qze
            plsc.store_scatter(relf_v, [base], rx)
            plsc.store_scatter(relf_v, [base + 1], ry)
            plsc.store_scatter(relf_v, [base + 2], rz)

        pltpu.async_copy(xw_hbm.at[sel_v], rows_v, sem).wait()
        pltpu.sync_copy(rows_v, gx_hbm.at[pl.ds(i * K, K)])
        pltpu.sync_copy(relf_v, rel_hbm.at[pl.ds(i * K * 4, K * 4)])
        return 0

    lax.fori_loop(0, QPW, _q, 0)


def _sc(px, py, pz, bat, xw):
    mesh = plsc.VectorSubcoreMesh(core_axis_name="c", subcore_axis_name="s")
    f = pl.kernel(
        _sc_body,
        out_type=[
            jax.ShapeDtypeStruct((N * K, DF), jnp.float32),
            jax.ShapeDtypeStruct((N * K * 4,), jnp.float32),
        ],
        mesh=mesh,
        compiler_params=pltpu.CompilerParams(
            needs_layout_passes=False, use_tc_tiling_on_sc=False),
        scratch_types=[
            pltpu.VMEM((N,), jnp.float32),
            pltpu.VMEM((N,), jnp.float32),
            pltpu.VMEM((N,), jnp.float32),
            pltpu.VMEM((N,), jnp.float32),
            pltpu.VMEM((N,), jnp.int32),
            pltpu.VMEM((N,), jnp.int32),
            pltpu.VMEM((N,), jnp.float32),
            pltpu.VMEM((34,), jnp.int32),
            pltpu.VMEM((HCAP,), jnp.float32),
            pltpu.VMEM((HCAP,), jnp.int32),
            pltpu.VMEM((48,), jnp.int32),
            pltpu.VMEM((K,), jnp.int32),
            pltpu.VMEM((K * 4,), jnp.float32),
            pltpu.VMEM((K, DF), jnp.float32),
            pltpu.SemaphoreType.DMA,
        ],
    )
    return f(px, py, pz, bat, xw)


QT = 256


def _mlp_body(gx_ref, rel_ref, w1r_ref, w2_ref, b2_ref, w3_ref, b3_ref, o_ref):
    h0 = gx_ref[...] + jnp.dot(rel_ref[...], w1r_ref[...],
                               preferred_element_type=jnp.float32)
    h1 = jnp.maximum(h0, 0.0)
    h2 = jnp.maximum(
        jnp.dot(h1, w2_ref[...], preferred_element_type=jnp.float32) + b2_ref[...], 0.0)
    h3 = jnp.maximum(
        jnp.dot(h2, w3_ref[...], preferred_element_type=jnp.float32) + b3_ref[...], 0.0)
    o_ref[...] = jnp.max(h3.reshape(QT, K, DO), axis=1)


def _mlp(gx, rel, w1r, w2, b2, w3, b3):
    return pl.pallas_call(
        _mlp_body,
        grid=(N // QT,),
        in_specs=[
            pl.BlockSpec((QT * K, DF), lambda i: (i, 0)),
            pl.BlockSpec((QT * K, 4), lambda i: (i, 0)),
            pl.BlockSpec((4, DF), lambda i: (0, 0)),
            pl.BlockSpec((DF, DF), lambda i: (0, 0)),
            pl.BlockSpec((1, DF), lambda i: (0, 0)),
            pl.BlockSpec((DF, DO), lambda i: (0, 0)),
            pl.BlockSpec((1, DO), lambda i: (0, 0)),
        ],
        out_specs=pl.BlockSpec((QT, DO), lambda i: (i, 0)),
        out_shape=jax.ShapeDtypeStruct((N, DO), jnp.float32),
    )(gx, rel, w1r, w2, b2, w3, b3)


@jax.jit
def kernel(x, pos, batch, W1, b1, W2, b2, W3, b3):
    xw = _xw(x, W1[:DF], b1.reshape(1, DF))
    gx, relf = _sc(pos[:, 0], pos[:, 1], pos[:, 2], batch, xw)
    w1r = jnp.concatenate([W1[DF:], jnp.zeros((1, DF), jnp.float32)], axis=0)
    out = _mlp(gx, relf.reshape(N * K, 4), w1r, W2,
               b2.reshape(1, DF), W3, b3.reshape(1, DO))
    return (out, pos, batch)


# speedup vs baseline: 25.9232x; 1.2910x over previous
import jax
import jax.numpy as jnp
from jax import lax
from jax.experimental import pallas as pl
from jax.experimental.pallas import tpu as pltpu
from jax.experimental.pallas import tpu_sc as plsc

N = 16384
DF = 64
DO = 128
K = 32
R2 = 0.2 * 0.2
NC = 2
NS = 16
NW = NC * NS
QPW = N // NW
HCAP = 512


def _xw_body(x_ref, w_ref, b_ref, o_ref):
    o_ref[...] = jnp.dot(x_ref[...], w_ref[...],
                         preferred_element_type=jnp.float32) + b_ref[...]


def _xw(x, w, b):
    t = 512
    return pl.pallas_call(
        _xw_body,
        grid=(N // t,),
        in_specs=[
            pl.BlockSpec((t, DF), lambda i: (i, 0)),
            pl.BlockSpec((DF, DF), lambda i: (0, 0)),
            pl.BlockSpec((1, DF), lambda i: (0, 0)),
        ],
        out_specs=pl.BlockSpec((t, DF), lambda i: (i, 0)),
        out_shape=jax.ShapeDtypeStruct((N, DF), jnp.float32),
    )(x, w, b)


def _sload(ref, idx):
    return plsc.load_gather(ref, [jnp.full((16,), idx, jnp.int32)])[0]


def _rbf16_bits(v):
    u = plsc.bitcast(v, jnp.int32)
    return (u + 0x7FFF + ((u >> 16) & 1)) & jnp.int32(-65536)


def _rbf16(v):
    return plsc.bitcast(_rbf16_bits(v), jnp.float32)


def _sc_body(px_hbm, py_hbm, pz_hbm, bat_hbm, xw_hbm, gx_hbm, rel_hbm,
             px_v, py_v, pz_v, p2_v, bat_v, pxy_v, pzp_v,
             seg_v, hd_v, hj_v, selb_v,
             sel0_v, sel1_v, sel2_v, relf0_v, relf1_v, relf2_v,
             rows0_v, rows1_v, rows2_v,
             semg0, semg1, semg2, semo0, semo1, semo2):
    wid = lax.axis_index("s") * NC + lax.axis_index("c")
    pltpu.sync_copy(px_hbm, px_v)
    pltpu.sync_copy(py_hbm, py_v)
    pltpu.sync_copy(pz_hbm, pz_v)
    pltpu.sync_copy(bat_hbm, bat_v)
    iot = lax.iota(jnp.int32, 16)

    zero16 = jnp.zeros((16,), jnp.float32)
    for rv in (relf0_v, relf1_v, relf2_v):
        for t in range(8):
            rv[pl.ds(t * 16, 16)] = zero16

    def _pre(c, _):
        s = c * 16
        vx = px_v[pl.ds(s, 16)]
        vy = py_v[pl.ds(s, 16)]
        vz = pz_v[pl.ds(s, 16)]
        p2_v[pl.ds(s, 16)] = (vx * vx + vy * vy) + vz * vz
        pxy_v[pl.ds(s, 16)] = _rbf16_bits(vx) | lax.shift_right_logical(
            _rbf16_bits(vy), 16)
        pzp_v[pl.ds(s, 16)] = _rbf16(vz)
        jg = iot + s
        bv = bat_v[pl.ds(s, 16)]
        prevv = plsc.load_gather(bat_v, [jnp.maximum(jg - 1, 0)])
        nxtv = plsc.load_gather(bat_v, [jnp.minimum(jg + 1, N - 1)])
        isf = (bv != prevv) | (jg == 0)
        isl = (bv != nxtv) | (jg == N - 1)
        plsc.store_scatter(seg_v, [bv], jg, mask=isf)
        plsc.store_scatter(seg_v, [bv + 16], jg + 1, mask=isl)
        return 0

    lax.fori_loop(0, N // 16, _pre, 0)

    q0 = wid * QPW

    def _scan_sel(i, sel_v, relf_v):
        vb = _sload(bat_v, i)
        lo = _sload(seg_v, vb)
        hi = _sload(seg_v, vb + 16)
        ii = jnp.full((16,), i, jnp.int32)
        qxv = plsc.load_gather(px_v, [ii])
        qyv = plsc.load_gather(py_v, [ii])
        qzv = plsc.load_gather(pz_v, [ii])
        qxe = qxv[0]
        qye = qyv[0]
        qze = qzv[0]
        q2 = _sload(p2_v, i)
        qw = plsc.load_gather(pxy_v, [ii])
        qx = plsc.bitcast(qw & jnp.int32(-65536), jnp.float32)[0]
        qy = plsc.bitcast(qw << 16, jnp.float32)[0]
        qz = plsc.load_gather(pzp_v, [ii])[0]
        c0 = lax.div(lo, 16)
        c1 = lax.div(hi + 15, 16)

        def _scan(c, cur):
            s = c * 16
            jg = iot + s
            w = pxy_v[pl.ds(s, 16)]
            vx = plsc.bitcast(w & jnp.int32(-65536), jnp.float32)
            vy = plsc.bitcast(w << 16, jnp.float32)
            vz = pzp_v[pl.ds(s, 16)]
            vp2 = p2_v[pl.ds(s, 16)]
            d2 = jnp.maximum((q2 + vp2) - 2.0 * ((qx * vx + qy * vy) + qz * vz), 0.0)
            m = (d2 <= R2) & (jg >= lo) & (jg < hi) & (cur < HCAP - 16)
            plsc.store_compressed(hd_v.at[pl.ds(cur, 16)], d2, mask=m)
            plsc.store_compressed(hj_v.at[pl.ds(cur, 16)], jg, mask=m)
            return cur + jnp.sum(m.astype(jnp.int32))

        nh = lax.fori_loop(c0, c1, _scan, 0)
        nch = lax.div(nh + 15, 16)

        def _cond(st):
            return st > K

        def _drop(st):
            def _mx(c, mx):
                dv = hd_v[pl.ds(c * 16, 16)]
                lm = (iot + c * 16) < nh
                return jnp.maximum(mx, jnp.max(jnp.where(lm, dv, -1.0)))

            mval = lax.fori_loop(0, nch, _mx, -1.0)

            def _pw(c, p):
                dv = hd_v[pl.ds(c * 16, 16)]
                jgl = iot + c * 16
                lm = (jgl < nh) & (dv == mval)
                return jnp.maximum(p, jnp.max(jnp.where(lm, jgl, -1)))

            ppos = lax.fori_loop(0, nch, _pw, -1)
            plsc.store_scatter(hd_v, [jnp.full((16,), ppos, jnp.int32)],
                               jnp.full((16,), -1.0, jnp.float32), mask=iot == 0)
            return st - 1

        lax.while_loop(_cond, _drop, nh)

        selb_v[pl.ds(0, 16)] = jnp.full((16,), i, jnp.int32)
        selb_v[pl.ds(16, 16)] = jnp.full((16,), i, jnp.int32)

        def _cp(c, cur):
            dv = hd_v[pl.ds(c * 16, 16)]
            jv = hj_v[pl.ds(c * 16, 16)]
            lm = ((iot + c * 16) < nh) & (dv >= 0.0)
            plsc.store_compressed(selb_v.at[pl.ds(cur, 16)], jv, mask=lm)
            return cur + jnp.sum(lm.astype(jnp.int32))

        lax.fori_loop(0, nch, _cp, 0)
        sel_v[pl.ds(0, 16)] = selb_v[pl.ds(0, 16)]
        sel_v[pl.ds(16, 16)] = selb_v[pl.ds(16, 16)]

        for h in range(2):
            selc = sel_v[pl.ds(h * 16, 16)]
            base = (iot + h * 16) * 4
            rx = plsc.load_gather(px_v, [selc]) - qxe
            ry = plsc.load_gather(py_v, [selc]) - qye
            rz = plsc.load_gather(pz_v, [selc]) - qze
            plsc.store_scatter(relf_v, [base], rx)
            plsc.store_scatter(relf_v, [base + 1], ry)
            plsc.store_scatter(relf_v, [base + 2], rz)

    S0 = (sel0_v, relf0_v, rows0_v, semg0, semo0)
    S1 = (sel1_v, relf1_v, rows1_v, semg1, semo1)
    S2 = (sel2_v, relf2_v, rows2_v, semg2, semo2)

    def _gather_start(S):
        pltpu.async_copy(xw_hbm.at[S[0]], S[2], S[3])

    def _gather_wait(S):
        pltpu.make_async_copy(xw_hbm.at[S[0]], S[2], S[3]).wait()

    def _out_start(i, S):
        pltpu.async_copy(S[2], gx_hbm.at[pl.ds(i * K, K)], S[4])
        pltpu.async_copy(S[1], rel_hbm.at[pl.ds(i * K * 4, K * 4)], S[4])

    def _out_wait(i, S):
        pltpu.make_async_copy(S[2], gx_hbm.at[pl.ds(i * K, K)], S[4]).wait()
        pltpu.make_async_copy(S[1], rel_hbm.at[pl.ds(i * K * 4, K * 4)],
                              S[4]).wait()

    def _step0(qi, P, O):
        _scan_sel(q0 + qi, P[0], P[1])
        _gather_start(P)
        _gather_wait(O)
        _out_start(q0 + qi - 1, O)

    def _step(qi, P, O):
        _out_wait(q0 + qi - 3, P)
        _step0(qi, P, O)

    _scan_sel(q0, S0[0], S0[1])
    _gather_start(S0)
    _step0(1, S1, S0)
    _step0(2, S2, S1)

    NT = (QPW - 5) // 3

    def _tri(ci, _):
        _step(3 * ci, S0, S2)
        _step(3 * ci + 1, S1, S0)
        _step(3 * ci + 2, S2, S1)
        return 0

    lax.fori_loop(1, NT + 1, _tri, 0)

    _step(QPW - 2, S0, S2)
    _step(QPW - 1, S1, S0)
    _gather_wait(S1)
    _out_start(q0 + QPW - 1, S1)
    _out_wait(q0 + QPW - 3, S2)
    _out_wait(q0 + QPW - 2, S0)
    _out_wait(q0 + QPW - 1, S1)


def _sc(px, py, pz, bat, xw):
    mesh = plsc.VectorSubcoreMesh(core_axis_name="c", subcore_axis_name="s")
    f = pl.kernel(
        _sc_body,
        out_type=[
            jax.ShapeDtypeStruct((N * K, DF), jnp.float32),
            jax.ShapeDtypeStruct((N * K * 4,), jnp.float32),
        ],
        mesh=mesh,
        compiler_params=pltpu.CompilerParams(
            needs_layout_passes=False, use_tc_tiling_on_sc=False),
        scratch_types=[
            pltpu.VMEM((N,), jnp.float32),
            pltpu.VMEM((N,), jnp.float32),
            pltpu.VMEM((N,), jnp.float32),
            pltpu.VMEM((N,), jnp.float32),
            pltpu.VMEM((N,), jnp.int32),
            pltpu.VMEM((N,), jnp.int32),
            pltpu.VMEM((N,), jnp.float32),
            pltpu.VMEM((34,), jnp.int32),
            pltpu.VMEM((HCAP,), jnp.float32),
            pltpu.VMEM((HCAP,), jnp.int32),
            pltpu.VMEM((48,), jnp.int32),
            pltpu.VMEM((K,), jnp.int32),
            pltpu.VMEM((K,), jnp.int32),
            pltpu.VMEM((K,), jnp.int32),
            pltpu.VMEM((K * 4,), jnp.float32),
            pltpu.VMEM((K * 4,), jnp.float32),
            pltpu.VMEM((K * 4,), jnp.float32),
            pltpu.VMEM((K, DF), jnp.float32),
            pltpu.VMEM((K, DF), jnp.float32),
            pltpu.VMEM((K, DF), jnp.float32),
            pltpu.SemaphoreType.DMA,
            pltpu.SemaphoreType.DMA,
            pltpu.SemaphoreType.DMA,
            pltpu.SemaphoreType.DMA,
            pltpu.SemaphoreType.DMA,
            pltpu.SemaphoreType.DMA,
        ],
    )
    return f(px, py, pz, bat, xw)


QT = 256


def _mlp_body(gx_ref, rel_ref, w1r_ref, w2_ref, b2_ref, w3_ref, b3_ref, o_ref):
    h0 = gx_ref[...] + jnp.dot(rel_ref[...], w1r_ref[...],
                               preferred_element_type=jnp.float32)
    h1 = jnp.maximum(h0, 0.0)
    h2 = jnp.maximum(
        jnp.dot(h1, w2_ref[...], preferred_element_type=jnp.float32) + b2_ref[...], 0.0)
    h3 = jnp.maximum(
        jnp.dot(h2, w3_ref[...], preferred_element_type=jnp.float32) + b3_ref[...], 0.0)
    o_ref[...] = jnp.max(h3.reshape(QT, K, DO), axis=1)


def _mlp(gx, rel, w1r, w2, b2, w3, b3):
    return pl.pallas_call(
        _mlp_body,
        grid=(N // QT,),
        in_specs=[
            pl.BlockSpec((QT * K, DF), lambda i: (i, 0)),
            pl.BlockSpec((QT * K, 4), lambda i: (i, 0)),
            pl.BlockSpec((4, DF), lambda i: (0, 0)),
            pl.BlockSpec((DF, DF), lambda i: (0, 0)),
            pl.BlockSpec((1, DF), lambda i: (0, 0)),
            pl.BlockSpec((DF, DO), lambda i: (0, 0)),
            pl.BlockSpec((1, DO), lambda i: (0, 0)),
        ],
        out_specs=pl.BlockSpec((QT, DO), lambda i: (i, 0)),
        out_shape=jax.ShapeDtypeStruct((N, DO), jnp.float32),
    )(gx, rel, w1r, w2, b2, w3, b3)


@jax.jit
def kernel(x, pos, batch, W1, b1, W2, b2, W3, b3):
    xw = _xw(x, W1[:DF], b1.reshape(1, DF))
    gx, relf = _sc(pos[:, 0], pos[:, 1], pos[:, 2], batch, xw)
    w1r = jnp.concatenate([W1[DF:], jnp.zeros((1, DF), jnp.float32)], axis=0)
    out = _mlp(gx, relf.reshape(N * K, 4), w1r, W2,
               b2.reshape(1, DF), W3, b3.reshape(1, DO))
    return (out, pos, batch)
